# baseline (speedup 1.0000x reference)
import functools

import jax
import jax.numpy as jnp
from jax import lax
from jax.experimental import pallas as pl
from jax.experimental.pallas import tpu as pltpu
from jax.experimental.pallas import tpu_sc as plsc

N = 2097152
NH = 2
H = N // NH
NW = 32
PER_W = H // NW
C = 16384
T = PER_W // C

MAX_CELL = 256.0 - 2.0 ** -15

ROWS = H // 128
BLK = 1024


def _idx_kernel(x_ref, o_ref):
    def cell(v):
        return jnp.minimum(v * 128.0 + 128.0, MAX_CELL).astype(jnp.int32)

    i = cell(x_ref[:, 0, :])
    j = cell(x_ref[:, 1, :])
    k = cell(x_ref[:, 2, :])
    jpart = (j << 7) + ((j >> 3) << 10)
    kpart = (k & 127) | ((k >> 7) << 10)
    o_ref[:, :] = (i << 16) | jpart | kpart


_compute_idx = pl.pallas_call(
    _idx_kernel,
    grid=(ROWS // BLK,),
    in_specs=[pl.BlockSpec((BLK, 3, 128), lambda g: (g, 0, 0))],
    out_specs=pl.BlockSpec((BLK, 128), lambda g: (g, 0)),
    out_shape=jax.ShapeDtypeStruct((ROWS, 128), jnp.int32),
)

_mesh = plsc.VectorSubcoreMesh(core_axis_name="c", subcore_axis_name="s")


@functools.partial(
    pl.kernel,
    mesh=_mesh,
    out_type=jax.ShapeDtypeStruct((H,), jnp.float32),
    compiler_params=pltpu.CompilerParams(needs_layout_passes=False),
    scratch_types=[
        pltpu.VMEM((C,), jnp.int32),
        pltpu.VMEM((C,), jnp.int32),
        pltpu.VMEM((C,), jnp.float32),
        pltpu.VMEM((C,), jnp.float32),
        pltpu.SemaphoreType.DMA,
        pltpu.SemaphoreType.DMA,
        pltpu.SemaphoreType.DMA,
        pltpu.SemaphoreType.DMA,
        pltpu.SemaphoreType.DMA,
        pltpu.SemaphoreType.DMA,
    ],
)
def _sc_gather(idx_hbm, grid_hbm, out_hbm,
               i0, i1, o0, o1, si0, si1, sg0, sg1, so0, so1):
    wid = lax.axis_index("s") * 2 + lax.axis_index("c")
    base = wid * PER_W
    ib = (i0, i1)
    ob = (o0, o1)
    si = (si0, si1)
    sg = (sg0, sg1)
    so = (so0, so1)

    def icopy(t):
        return pltpu.make_async_copy(
            idx_hbm.at[pl.ds(base + t * C, C)], ib[t % 2], si[t % 2])

    def gcopy(t):
        return pltpu.make_async_copy(grid_hbm.at[ib[t % 2]], ob[t % 2], sg[t % 2])

    def ocopy(t):
        return pltpu.make_async_copy(
            ob[t % 2], out_hbm.at[pl.ds(base + t * C, C)], so[t % 2])

    icopy(0).start()
    icopy(1).start()
    for t in range(T):
        icopy(t).wait()
        if t >= 2:
            ocopy(t - 2).wait()
        gcopy(t).start()
        gcopy(t).wait()
        if t + 2 < T:
            icopy(t + 2).start()
        ocopy(t).start()
    ocopy(T - 2).wait()
    ocopy(T - 1).wait()


def kernel(x, grid):
    grid_lin = (
        grid.reshape(256, 32, 8, 2, 128)
        .transpose(0, 1, 3, 2, 4)
        .reshape(-1)
    )
    xr = x.reshape(NH, H // 128, 128, 3)
    halves = []
    for h in range(NH):
        xt = xr[h].transpose(0, 2, 1)
        idx = _compute_idx(xt).reshape(-1)
        halves.append(_sc_gather(idx, grid_lin))
    return jnp.concatenate(halves)


# speedup vs baseline: 1.0356x; 1.0356x over previous
import functools

import jax
import jax.numpy as jnp
from jax import lax
from jax.experimental import pallas as pl
from jax.experimental.pallas import tpu as pltpu
from jax.experimental.pallas import tpu_sc as plsc

N = 2097152
NW = 32
PER_W = N // NW
C = 16384
T = PER_W // C

MAX_CELL = 256.0 - 2.0 ** -15

ROWS = N // 128
BLK = 1024


def _idx_kernel(x_ref, o_ref):
    def cell(v):
        return jnp.minimum(v * 128.0 + 128.0, MAX_CELL).astype(jnp.int32)

    i = cell(x_ref[:, 0, :])
    j = cell(x_ref[:, 1, :])
    k = cell(x_ref[:, 2, :])
    jpart = (j << 7) + ((j >> 3) << 10)
    kpart = (k & 127) | ((k >> 7) << 10)
    o_ref[:, :] = (i << 16) | jpart | kpart


_compute_idx = pl.pallas_call(
    _idx_kernel,
    grid=(ROWS // BLK,),
    in_specs=[pl.BlockSpec((BLK, 3, 128), lambda g: (g, 0, 0))],
    out_specs=pl.BlockSpec((BLK, 128), lambda g: (g, 0)),
    out_shape=jax.ShapeDtypeStruct((ROWS, 128), jnp.int32),
)

_mesh = plsc.VectorSubcoreMesh(core_axis_name="c", subcore_axis_name="s")


@functools.partial(
    pl.kernel,
    mesh=_mesh,
    out_type=jax.ShapeDtypeStruct((N,), jnp.float32),
    compiler_params=pltpu.CompilerParams(needs_layout_passes=False),
    scratch_types=[
        pltpu.VMEM((C,), jnp.int32),
        pltpu.VMEM((C,), jnp.int32),
        pltpu.VMEM((C,), jnp.float32),
        pltpu.VMEM((C,), jnp.float32),
        pltpu.SemaphoreType.DMA,
        pltpu.SemaphoreType.DMA,
        pltpu.SemaphoreType.DMA,
        pltpu.SemaphoreType.DMA,
        pltpu.SemaphoreType.DMA,
        pltpu.SemaphoreType.DMA,
    ],
)
def _sc_gather(idx_hbm, grid_hbm, out_hbm,
               i0, i1, o0, o1, si0, si1, sg0, sg1, so0, so1):
    wid = lax.axis_index("s") * 2 + lax.axis_index("c")
    base = wid * PER_W
    ib = (i0, i1)
    ob = (o0, o1)
    si = (si0, si1)
    sg = (sg0, sg1)
    so = (so0, so1)

    def icopy(t):
        return pltpu.make_async_copy(
            idx_hbm.at[pl.ds(base + t * C, C)], ib[t % 2], si[t % 2])

    def gcopy(t):
        return pltpu.make_async_copy(grid_hbm.at[ib[t % 2]], ob[t % 2], sg[t % 2])

    def ocopy(t):
        return pltpu.make_async_copy(
            ob[t % 2], out_hbm.at[pl.ds(base + t * C, C)], so[t % 2])

    icopy(0).start()
    icopy(1).start()
    for t in range(T):
        icopy(t).wait()
        if t >= 2:
            ocopy(t - 2).wait()
        gcopy(t).start()
        gcopy(t).wait()
        if t + 2 < T:
            icopy(t + 2).start()
        ocopy(t).start()
    ocopy(T - 2).wait()
    ocopy(T - 1).wait()


def kernel(x, grid):
    grid_lin = (
        grid.reshape(256, 32, 8, 2, 128)
        .transpose(0, 1, 3, 2, 4)
        .reshape(-1)
    )
    xt = x.reshape(-1, 128, 3).transpose(0, 2, 1)
    idx = _compute_idx(xt).reshape(-1)
    return _sc_gather(idx, grid_lin)
